# baseline (speedup 1.0000x reference)
import functools

import jax
import jax.numpy as jnp
from jax import lax
from jax.experimental import pallas as pl
from jax.experimental.pallas import tpu as pltpu
from jax.experimental.pallas import tpu_sc as plsc

NC = 2
NS = 16
NW = NC * NS
C = 128
NB = 3
LA = 2
LANES = 128


def _agg_body(Q0, Q1, REM, RPT, NT, D, MODE, g_hbm, ei3, z_hbm, out_hbm,
              acc, tbl, src_v, dst_v, *bufs):
  rows = bufs[:NB]
  gsems = bufs[NB:2 * NB]
  ssems = bufs[2 * NB:3 * NB]
  ones_v = bufs[3 * NB] if MODE == "const" else None
  cid = lax.axis_index("c")
  sid = lax.axis_index("s")
  wid = sid * NC + cid
  qmax = max(Q0, Q1)
  base = jnp.where(cid == 0, sid * Q0, NS * Q0 + sid * Q1)
  ebase = NS * (Q0 + Q1)

  @pl.when(cid == 0)
  def _():
    pltpu.sync_copy(ei3.at[0, pl.ds(base, Q0)], src_v.at[pl.ds(0, Q0)])
    pltpu.sync_copy(ei3.at[1, pl.ds(base, Q0)], dst_v.at[pl.ds(0, Q0)])

  @pl.when(cid == 1)
  def _():
    pltpu.sync_copy(ei3.at[0, pl.ds(base, Q1)], src_v.at[pl.ds(0, Q1)])
    pltpu.sync_copy(ei3.at[1, pl.ds(base, Q1)], dst_v.at[pl.ds(0, Q1)])

  @pl.when(wid < REM)
  def _():
    pltpu.sync_copy(ei3.at[0, pl.ds(ebase + wid, 1)],
                    src_v.at[pl.ds(qmax, 1)])
    pltpu.sync_copy(ei3.at[1, pl.ds(ebase + wid, 1)],
                    dst_v.at[pl.ds(qmax, 1)])

  pltpu.sync_copy(z_hbm.at[pl.ds(sid * RPT, RPT)],
                  acc.at[pl.ds(sid * RPT, RPT)])

  if MODE == "const":
    pltpu.sync_copy(g_hbm, ones_v)
  else:
    stage = (cid >= 0)
    last = NT - (NS - 1) * RPT

    @pl.when(jnp.logical_and(stage, sid < NS - 1))
    def _():
      pltpu.sync_copy(g_hbm.at[pl.ds(sid * RPT, RPT)],
                      tbl.at[pl.ds(sid * RPT, RPT)])

    @pl.when(jnp.logical_and(stage, sid == NS - 1))
    def _():
      pltpu.sync_copy(g_hbm.at[pl.ds((NS - 1) * RPT, last)],
                      tbl.at[pl.ds((NS - 1) * RPT, last)])

  plsc.subcore_barrier()

  if MODE == "const":
    def cstep(c0, carry):
      for b in range(NB):
        c = c0 * NB + b

        @pl.when(c >= NB)
        def _():
          pltpu.make_async_copy(ones_v, acc.at[dst_v.at[c - NB]],
                                ssems[b]).wait()

        pltpu.async_copy(ones_v, acc.at[dst_v.at[c]], ssems[b], add=True)
      return carry

    lax.fori_loop(0, Q0 // NB, cstep, 0)
    for b in range(NB):
      pltpu.make_async_copy(ones_v, acc.at[dst_v.at[Q0 - NB + b]],
                            ssems[b]).wait()

    @pl.when(wid < REM)
    def _():
      pltpu.sync_copy(ones_v, acc.at[dst_v.at[qmax]], add=True)

  else:
    def pipeline(srcs, q):
      for j in range(LA):
        pltpu.async_copy(srcs[j].at[src_v.at[j]], rows[j], gsems[j])

      def step(c0, carry):
        for b in range(NB):
          c = c0 * NB + b
          f = c + LA
          bf = (b + LA) % NB

          @pl.when(jnp.logical_and(f < q, f >= NB))
          def _():
            pltpu.make_async_copy(rows[bf], acc.at[dst_v.at[f - NB]],
                                  ssems[bf]).wait()

          @pl.when(f < q)
          def _():
            pltpu.async_copy(srcs[bf].at[src_v.at[f]], rows[bf],
                             gsems[bf])

          pltpu.make_async_copy(srcs[b].at[src_v.at[c]], rows[b],
                                gsems[b]).wait()
          pltpu.async_copy(rows[b], acc.at[dst_v.at[c]], ssems[b],
                           add=True)
        return carry

      lax.fori_loop(0, q // NB, step, 0)
      for b in range(NB):
        pltpu.make_async_copy(rows[b], acc.at[dst_v.at[q - NB + b]],
                              ssems[b]).wait()

      @pl.when(wid < REM)
      def _():
        pltpu.async_copy(srcs[0].at[src_v.at[qmax]], rows[0],
                         gsems[0]).wait()
        pltpu.sync_copy(rows[0], acc.at[dst_v.at[qmax]], add=True)

    if MODE == "hybrid":
      @pl.when(cid == 0)
      def _():
        pipeline([g_hbm if b % 2 == 0 else tbl for b in range(NB)], Q0)

      @pl.when(cid == 1)
      def _():
        pipeline([tbl] * NB, Q1)
    else:
      pipeline([tbl] * NB, Q0)

  plsc.subcore_barrier()
  pltpu.sync_copy(acc.at[pl.ds(sid * RPT, RPT)],
                  out_hbm.at[pl.ds(sid * RPT, RPT), pl.ds(cid * D, D)])


@functools.cache
def _make_agg(d, q0, q1, rem, nrows, nt, mode):
  rpt = nrows // NS
  qmax = max(q0, q1)
  mesh = plsc.VectorSubcoreMesh(core_axis_name="c", subcore_axis_name="s",
                                num_cores=NC, num_subcores=NS)
  scratch = [
      pltpu.VMEM_SHARED((nrows, d), jnp.float32),
      pltpu.VMEM_SHARED((nt, d), jnp.float32),
      pltpu.VMEM((qmax + 1, C), jnp.int32),
      pltpu.VMEM((qmax + 1, C), jnp.int32),
      *[pltpu.VMEM((C, d), jnp.float32) for _ in range(NB)],
      *[pltpu.SemaphoreType.DMA for _ in range(2 * NB)],
  ]
  if mode == "const":
    scratch.append(pltpu.VMEM((C, d), jnp.float32))
  return pl.kernel(
      functools.partial(_agg_body, q0, q1, rem, rpt, nt, d, mode),
      out_type=jax.ShapeDtypeStruct((nrows, LANES), jnp.float32),
      mesh=mesh,
      compiler_params=pltpu.CompilerParams(use_tc_tiling_on_sc=False),
      scratch_types=scratch,
  )


def _tc_mm(x_ref, w_ref, p_ref):
  p_ref[...] = jnp.dot(x_ref[...], w_ref[...],
                       preferred_element_type=jnp.float32)


def _tc_scale(n, d3, dp_ref, p_ref, g_ref, dinv_ref):
  deg = dp_ref[:n, 0:1] + dp_ref[:n, d3:d3 + 1] + 1.0
  dinv = lax.rsqrt(jnp.maximum(deg, 1.0))
  g_ref[...] = dinv * p_ref[...]
  dinv_ref[...] = dinv


def _tc_mid(n, dh, ap_ref, g_ref, dinv_ref, b_ref, w_ref, gout_ref):
  dinv = dinv_ref[...]
  t = (dinv * (ap_ref[:n, :dh] + ap_ref[:n, dh:2 * dh] + g_ref[...])
       + b_ref[...])
  t = jnp.maximum(t, 0.0)
  gout_ref[...] = dinv * jnp.dot(t, w_ref[...],
                                 preferred_element_type=jnp.float32)


def _tc_tail(n, d3, do, ap_ref, g_ref, dinv_ref, b_ref, out_ref):
  t = (dinv_ref[...] * (ap_ref[:n, :d3] + ap_ref[:n, d3:2 * d3]
                        + g_ref[...]) + b_ref[...])
  out_ref[...] = t[:, :do]


def _splits(tchunks, ratio):
  s = tchunks // NS
  q0 = int(round(s * ratio / (1.0 + ratio) / NB)) * NB
  q0 = max(NB, min(q0, s - NB))
  q1 = ((s - q0) // NB) * NB
  rem = tchunks - NS * (q0 + q1)
  assert 0 <= rem <= NW and q0 >= NB and q1 >= NB, (tchunks, q0, q1, rem)
  return q0, q1, rem


def kernel(x, edge_index, W1, b1, W2, b2, W3, b3):
  n, d_in = x.shape
  e = edge_index.shape[1]
  dh = W1.shape[1]
  do = W3.shape[1]
  d3 = 8

  if e % C == 0:
    ei = edge_index
  else:
    tail = C - e % C
    ei = jnp.concatenate(
        [edge_index,
         jnp.stack([jnp.zeros((tail,), edge_index.dtype),
                    jnp.full((tail,), n, edge_index.dtype)])], axis=1)
  tchunks = ei.shape[1] // C
  ei3 = ei.reshape(2, tchunks, C)

  q0h, q1h, remh = _splits(tchunks, 1.74)
  q0s, q1s, rems = _splits(tchunks, 1.0)

  nrows = -(-(n + 1) // (NS * 8)) * (NS * 8)

  z64 = jnp.zeros((nrows, dh), jnp.float32)
  z8 = jnp.zeros((nrows, d3), jnp.float32)
  ones8 = jnp.ones((C, d3), jnp.float32)
  W3p = jnp.concatenate([W3, jnp.zeros((dh, d3 - do), W3.dtype)], axis=1)
  b3p = jnp.concatenate([b3, jnp.zeros((d3 - do,), b3.dtype)])

  agg64 = _make_agg(dh, q0h, q1h, remh, nrows, n, "hybrid")
  agg8 = _make_agg(d3, q0s, q1s, rems, nrows, n, "spmem")
  deg8 = _make_agg(d3, q0s, q1s, rems, nrows, n, "const")

  degp = deg8(ones8, ei3, z8)
  p1 = pl.pallas_call(
      _tc_mm, out_shape=jax.ShapeDtypeStruct((n, dh), jnp.float32),
  )(x, W1)

  g1, dinv = pl.pallas_call(
      functools.partial(_tc_scale, n, d3),
      out_shape=(jax.ShapeDtypeStruct((n, dh), jnp.float32),
                 jax.ShapeDtypeStruct((n, 1), jnp.float32)),
  )(degp, p1)

  a1 = agg64(g1, ei3, z64)
  g2 = pl.pallas_call(
      functools.partial(_tc_mid, n, dh),
      out_shape=jax.ShapeDtypeStruct((n, dh), jnp.float32),
  )(a1, g1, dinv, b1.reshape(1, dh), W2)

  a2 = agg64(g2, ei3, z64)
  g3 = pl.pallas_call(
      functools.partial(_tc_mid, n, dh),
      out_shape=jax.ShapeDtypeStruct((n, d3), jnp.float32),
  )(a2, g2, dinv, b2.reshape(1, dh), W3p)

  a3 = agg8(g3, ei3, z8)
  dx = pl.pallas_call(
      functools.partial(_tc_tail, n, d3, do),
      out_shape=jax.ShapeDtypeStruct((n, do), jnp.float32),
  )(a3, g3, dinv, b3p.reshape(1, d3))

  return dx


# speedup vs baseline: 1.2321x; 1.2321x over previous
import functools

import jax
import jax.numpy as jnp
from jax import lax
from jax.experimental import pallas as pl
from jax.experimental.pallas import tpu as pltpu
from jax.experimental.pallas import tpu_sc as plsc

NC = 2
NS = 16
NW = NC * NS
C = 128
NB = 3
LA = 2
LANES = 128


def _agg_body(Q0, Q1, REM, RPT, NT, D, MODE, g_hbm, ei3, z_hbm, out_hbm,
              acc, tbl, src_v, dst_v, *bufs):
  rows = bufs[:NB]
  gsems = bufs[NB:2 * NB]
  ssems = bufs[2 * NB:3 * NB]
  ones_v = bufs[3 * NB] if MODE == "const" else None
  cid = lax.axis_index("c")
  sid = lax.axis_index("s")
  wid = sid * NC + cid
  qmax = max(Q0, Q1)
  base = jnp.where(cid == 0, sid * Q0, NS * Q0 + sid * Q1)
  ebase = NS * (Q0 + Q1)

  @pl.when(cid == 0)
  def _():
    pltpu.sync_copy(ei3.at[0, pl.ds(base, Q0)], src_v.at[pl.ds(0, Q0)])
    pltpu.sync_copy(ei3.at[1, pl.ds(base, Q0)], dst_v.at[pl.ds(0, Q0)])

  @pl.when(cid == 1)
  def _():
    pltpu.sync_copy(ei3.at[0, pl.ds(base, Q1)], src_v.at[pl.ds(0, Q1)])
    pltpu.sync_copy(ei3.at[1, pl.ds(base, Q1)], dst_v.at[pl.ds(0, Q1)])

  @pl.when(wid < REM)
  def _():
    pltpu.sync_copy(ei3.at[0, pl.ds(ebase + wid, 1)],
                    src_v.at[pl.ds(qmax, 1)])
    pltpu.sync_copy(ei3.at[1, pl.ds(ebase + wid, 1)],
                    dst_v.at[pl.ds(qmax, 1)])

  pltpu.sync_copy(z_hbm.at[pl.ds(sid * RPT, RPT)],
                  acc.at[pl.ds(sid * RPT, RPT)])

  if MODE == "const":
    pltpu.sync_copy(g_hbm, ones_v)
  else:
    stage = (cid == 1) if MODE == "hybrid" else (cid >= 0)
    last = NT - (NS - 1) * RPT

    @pl.when(jnp.logical_and(stage, sid < NS - 1))
    def _():
      pltpu.sync_copy(g_hbm.at[pl.ds(sid * RPT, RPT)],
                      tbl.at[pl.ds(sid * RPT, RPT)])

    @pl.when(jnp.logical_and(stage, sid == NS - 1))
    def _():
      pltpu.sync_copy(g_hbm.at[pl.ds((NS - 1) * RPT, last)],
                      tbl.at[pl.ds((NS - 1) * RPT, last)])

  plsc.subcore_barrier()

  if MODE == "const":
    def cstep(c0, carry):
      for b in range(NB):
        c = c0 * NB + b

        @pl.when(c >= NB)
        def _():
          pltpu.make_async_copy(ones_v, acc.at[dst_v.at[c - NB]],
                                ssems[b]).wait()

        pltpu.async_copy(ones_v, acc.at[dst_v.at[c]], ssems[b], add=True)
      return carry

    lax.fori_loop(0, Q0 // NB, cstep, 0)
    for b in range(NB):
      pltpu.make_async_copy(ones_v, acc.at[dst_v.at[Q0 - NB + b]],
                            ssems[b]).wait()

    @pl.when(wid < REM)
    def _():
      pltpu.sync_copy(ones_v, acc.at[dst_v.at[qmax]], add=True)

  else:
    def pipeline(srcs, q):
      for j in range(LA):
        pltpu.async_copy(srcs[j].at[src_v.at[j]], rows[j], gsems[j])

      def step(c0, carry):
        for b in range(NB):
          c = c0 * NB + b
          f = c + LA
          bf = (b + LA) % NB

          @pl.when(jnp.logical_and(f < q, f >= NB))
          def _():
            pltpu.make_async_copy(rows[bf], acc.at[dst_v.at[f - NB]],
                                  ssems[bf]).wait()

          @pl.when(f < q)
          def _():
            pltpu.async_copy(srcs[bf].at[src_v.at[f]], rows[bf],
                             gsems[bf])

          pltpu.make_async_copy(srcs[b].at[src_v.at[c]], rows[b],
                                gsems[b]).wait()
          pltpu.async_copy(rows[b], acc.at[dst_v.at[c]], ssems[b],
                           add=True)
        return carry

      lax.fori_loop(0, q // NB, step, 0)
      for b in range(NB):
        pltpu.make_async_copy(rows[b], acc.at[dst_v.at[q - NB + b]],
                              ssems[b]).wait()

      @pl.when(wid < REM)
      def _():
        pltpu.async_copy(srcs[0].at[src_v.at[qmax]], rows[0],
                         gsems[0]).wait()
        pltpu.sync_copy(rows[0], acc.at[dst_v.at[qmax]], add=True)

    if MODE == "hybrid":
      @pl.when(cid == 0)
      def _():
        pipeline([g_hbm] * NB, Q0)

      @pl.when(cid == 1)
      def _():
        pipeline([tbl] * NB, Q1)
    else:
      pipeline([tbl] * NB, Q0)

  plsc.subcore_barrier()
  pltpu.sync_copy(acc.at[pl.ds(sid * RPT, RPT)],
                  out_hbm.at[pl.ds(sid * RPT, RPT), pl.ds(cid * D, D)])


@functools.cache
def _make_agg(d, q0, q1, rem, nrows, nt, mode):
  rpt = nrows // NS
  qmax = max(q0, q1)
  mesh = plsc.VectorSubcoreMesh(core_axis_name="c", subcore_axis_name="s",
                                num_cores=NC, num_subcores=NS)
  scratch = [
      pltpu.VMEM_SHARED((nrows, d), jnp.float32),
      pltpu.VMEM_SHARED((nt, d), jnp.float32),
      pltpu.VMEM((qmax + 1, C), jnp.int32),
      pltpu.VMEM((qmax + 1, C), jnp.int32),
      *[pltpu.VMEM((C, d), jnp.float32) for _ in range(NB)],
      *[pltpu.SemaphoreType.DMA for _ in range(2 * NB)],
  ]
  if mode == "const":
    scratch.append(pltpu.VMEM((C, d), jnp.float32))
  return pl.kernel(
      functools.partial(_agg_body, q0, q1, rem, rpt, nt, d, mode),
      out_type=jax.ShapeDtypeStruct((nrows, LANES), jnp.float32),
      mesh=mesh,
      compiler_params=pltpu.CompilerParams(use_tc_tiling_on_sc=False),
      scratch_types=scratch,
  )


def _tc_mm(x_ref, w_ref, p_ref):
  p_ref[...] = jnp.dot(x_ref[...], w_ref[...],
                       preferred_element_type=jnp.float32)


def _tc_scale(n, d3, dp_ref, p_ref, g_ref, dinv_ref):
  deg = dp_ref[:n, 0:1] + dp_ref[:n, d3:d3 + 1] + 1.0
  dinv = lax.rsqrt(jnp.maximum(deg, 1.0))
  g_ref[...] = dinv * p_ref[...]
  dinv_ref[...] = dinv


def _tc_mid(n, dh, ap_ref, g_ref, dinv_ref, b_ref, w_ref, gout_ref):
  dinv = dinv_ref[...]
  t = (dinv * (ap_ref[:n, :dh] + ap_ref[:n, dh:2 * dh] + g_ref[...])
       + b_ref[...])
  t = jnp.maximum(t, 0.0)
  gout_ref[...] = dinv * jnp.dot(t, w_ref[...],
                                 preferred_element_type=jnp.float32)


def _tc_tail(n, d3, do, ap_ref, g_ref, dinv_ref, b_ref, out_ref):
  t = (dinv_ref[...] * (ap_ref[:n, :d3] + ap_ref[:n, d3:2 * d3]
                        + g_ref[...]) + b_ref[...])
  out_ref[...] = t[:, :do]


def _splits(tchunks, ratio):
  s = tchunks // NS
  q0 = int(round(s * ratio / (1.0 + ratio) / NB)) * NB
  q0 = max(NB, min(q0, s - NB))
  q1 = ((s - q0) // NB) * NB
  rem = tchunks - NS * (q0 + q1)
  assert 0 <= rem <= NW and q0 >= NB and q1 >= NB, (tchunks, q0, q1, rem)
  return q0, q1, rem


def kernel(x, edge_index, W1, b1, W2, b2, W3, b3):
  n, d_in = x.shape
  e = edge_index.shape[1]
  dh = W1.shape[1]
  do = W3.shape[1]
  d3 = 8

  if e % C == 0:
    ei = edge_index
  else:
    tail = C - e % C
    ei = jnp.concatenate(
        [edge_index,
         jnp.stack([jnp.zeros((tail,), edge_index.dtype),
                    jnp.full((tail,), n, edge_index.dtype)])], axis=1)
  tchunks = ei.shape[1] // C
  ei3 = ei.reshape(2, tchunks, C)

  q0h, q1h, remh = _splits(tchunks, 1.55)
  q0s, q1s, rems = _splits(tchunks, 1.0)

  nrows = -(-(n + 1) // (NS * 8)) * (NS * 8)

  z64 = jnp.zeros((nrows, dh), jnp.float32)
  z8 = jnp.zeros((nrows, d3), jnp.float32)
  ones8 = jnp.ones((C, d3), jnp.float32)
  W3p = jnp.concatenate([W3, jnp.zeros((dh, d3 - do), W3.dtype)], axis=1)
  b3p = jnp.concatenate([b3, jnp.zeros((d3 - do,), b3.dtype)])

  agg64 = _make_agg(dh, q0h, q1h, remh, nrows, n, "hybrid")
  agg8 = _make_agg(d3, q0s, q1s, rems, nrows, n, "spmem")
  deg8 = _make_agg(d3, q0s, q1s, rems, nrows, n, "const")

  degp = deg8(ones8, ei3, z8)
  p1 = pl.pallas_call(
      _tc_mm, out_shape=jax.ShapeDtypeStruct((n, dh), jnp.float32),
  )(x, W1)

  g1, dinv = pl.pallas_call(
      functools.partial(_tc_scale, n, d3),
      out_shape=(jax.ShapeDtypeStruct((n, dh), jnp.float32),
                 jax.ShapeDtypeStruct((n, 1), jnp.float32)),
  )(degp, p1)

  a1 = agg64(g1, ei3, z64)
  g2 = pl.pallas_call(
      functools.partial(_tc_mid, n, dh),
      out_shape=jax.ShapeDtypeStruct((n, dh), jnp.float32),
  )(a1, g1, dinv, b1.reshape(1, dh), W2)

  a2 = agg64(g2, ei3, z64)
  g3 = pl.pallas_call(
      functools.partial(_tc_mid, n, dh),
      out_shape=jax.ShapeDtypeStruct((n, d3), jnp.float32),
  )(a2, g2, dinv, b2.reshape(1, dh), W3p)

  a3 = agg8(g3, ei3, z8)
  dx = pl.pallas_call(
      functools.partial(_tc_tail, n, d3, do),
      out_shape=jax.ShapeDtypeStruct((n, do), jnp.float32),
  )(a3, g3, dinv, b3p.reshape(1, d3))

  return dx


# speedup vs baseline: 1.2444x; 1.0100x over previous
import functools

import jax
import jax.numpy as jnp
from jax import lax
from jax.experimental import pallas as pl
from jax.experimental.pallas import tpu as pltpu
from jax.experimental.pallas import tpu_sc as plsc

NC = 2
NS = 16
NW = NC * NS
C = 128
NB = 3
LA = 2
LANES = 128


def _agg_body(Q0, Q1, REM, RPT, NT, D, MODE, CW, g_hbm, ei3, z_hbm, out_hbm,
              acc, tbl, src_v, dst_v, *bufs):
  rows = bufs[:NB]
  gsems = bufs[NB:2 * NB]
  ssems = bufs[2 * NB:3 * NB]
  ones_v = bufs[3 * NB] if MODE == "const" else None
  cid = lax.axis_index("c")
  sid = lax.axis_index("s")
  wid = sid * NC + cid
  qmax = max(Q0, Q1)
  base = jnp.where(cid == 0, sid * Q0, NS * Q0 + sid * Q1)
  ebase = NS * (Q0 + Q1)

  @pl.when(cid == 0)
  def _():
    pltpu.sync_copy(ei3.at[0, pl.ds(base, Q0)], src_v.at[pl.ds(0, Q0)])
    pltpu.sync_copy(ei3.at[1, pl.ds(base, Q0)], dst_v.at[pl.ds(0, Q0)])

  @pl.when(cid == 1)
  def _():
    pltpu.sync_copy(ei3.at[0, pl.ds(base, Q1)], src_v.at[pl.ds(0, Q1)])
    pltpu.sync_copy(ei3.at[1, pl.ds(base, Q1)], dst_v.at[pl.ds(0, Q1)])

  @pl.when(wid < REM)
  def _():
    pltpu.sync_copy(ei3.at[0, pl.ds(ebase + wid, 1)],
                    src_v.at[pl.ds(qmax, 1)])
    pltpu.sync_copy(ei3.at[1, pl.ds(ebase + wid, 1)],
                    dst_v.at[pl.ds(qmax, 1)])

  pltpu.sync_copy(z_hbm.at[pl.ds(sid * RPT, RPT)],
                  acc.at[pl.ds(sid * RPT, RPT)])

  if MODE == "const":
    pltpu.sync_copy(g_hbm, ones_v)
  else:
    stage = (cid == 1) if MODE == "hybrid" else (cid >= 0)
    last = NT - (NS - 1) * RPT

    @pl.when(jnp.logical_and(stage, sid < NS - 1))
    def _():
      pltpu.sync_copy(g_hbm.at[pl.ds(sid * RPT, RPT)],
                      tbl.at[pl.ds(sid * RPT, RPT)])

    @pl.when(jnp.logical_and(stage, sid == NS - 1))
    def _():
      pltpu.sync_copy(g_hbm.at[pl.ds((NS - 1) * RPT, last)],
                      tbl.at[pl.ds((NS - 1) * RPT, last)])

  plsc.subcore_barrier()

  if MODE == "const":
    def cstep(c0, carry):
      for b in range(NB):
        c = c0 * NB + b

        @pl.when(c >= NB)
        def _():
          pltpu.make_async_copy(ones_v, acc.at[dst_v.at[c - NB]],
                                ssems[b]).wait()

        pltpu.async_copy(ones_v, acc.at[dst_v.at[c]], ssems[b], add=True)
      return carry

    lax.fori_loop(0, Q0 // NB, cstep, 0)
    for b in range(NB):
      pltpu.make_async_copy(ones_v, acc.at[dst_v.at[Q0 - NB + b]],
                            ssems[b]).wait()

    @pl.when(wid < REM)
    def _():
      pltpu.sync_copy(ones_v, acc.at[dst_v.at[qmax]], add=True)

  else:
    def pipeline(srcs, q):
      for j in range(LA):
        pltpu.async_copy(srcs[j].at[src_v.at[j]], rows[j], gsems[j])

      def step(c0, carry):
        for b in range(NB):
          c = c0 * NB + b
          f = c + LA
          bf = (b + LA) % NB

          @pl.when(jnp.logical_and(f < q, f >= NB))
          def _():
            pltpu.make_async_copy(rows[bf], acc.at[dst_v.at[f - NB]],
                                  ssems[bf]).wait()

          @pl.when(f < q)
          def _():
            pltpu.async_copy(srcs[bf].at[src_v.at[f]], rows[bf],
                             gsems[bf])

          pltpu.make_async_copy(srcs[b].at[src_v.at[c]], rows[b],
                                gsems[b]).wait()
          pltpu.async_copy(rows[b], acc.at[dst_v.at[c]], ssems[b],
                           add=True)
        return carry

      lax.fori_loop(0, q // NB, step, 0)
      for b in range(NB):
        pltpu.make_async_copy(rows[b], acc.at[dst_v.at[q - NB + b]],
                              ssems[b]).wait()

      @pl.when(wid < REM)
      def _():
        pltpu.async_copy(srcs[0].at[src_v.at[qmax]], rows[0],
                         gsems[0]).wait()
        pltpu.sync_copy(rows[0], acc.at[dst_v.at[qmax]], add=True)

    if MODE == "hybrid":
      @pl.when(cid == 0)
      def _():
        pipeline([g_hbm] * NB, Q0)

      @pl.when(cid == 1)
      def _():
        pipeline([tbl] * NB, Q1)
    else:
      pipeline([tbl] * NB, Q0)

  plsc.subcore_barrier()
  pltpu.sync_copy(acc.at[pl.ds(sid * RPT, RPT)],
                  out_hbm.at[pl.ds(sid * RPT, RPT), pl.ds(cid * D, D)])


@functools.cache
def _make_agg(d, q0, q1, rem, nrows, nt, mode, cw):
  rpt = nrows // NS
  qmax = max(q0, q1)
  assert q0 % NB == 0 and q1 % NB == 0, (q0, q1)
  mesh = plsc.VectorSubcoreMesh(core_axis_name="c", subcore_axis_name="s",
                                num_cores=NC, num_subcores=NS)
  scratch = [
      pltpu.VMEM_SHARED((nrows, d), jnp.float32),
      pltpu.VMEM_SHARED((nt, d), jnp.float32),
      pltpu.VMEM((qmax + 1, cw), jnp.int32),
      pltpu.VMEM((qmax + 1, cw), jnp.int32),
      *[pltpu.VMEM((cw, d), jnp.float32) for _ in range(NB)],
      *[pltpu.SemaphoreType.DMA for _ in range(2 * NB)],
  ]
  if mode == "const":
    scratch.append(pltpu.VMEM((cw, d), jnp.float32))
  return pl.kernel(
      functools.partial(_agg_body, q0, q1, rem, rpt, nt, d, mode, cw),
      out_type=jax.ShapeDtypeStruct((nrows, LANES), jnp.float32),
      mesh=mesh,
      compiler_params=pltpu.CompilerParams(use_tc_tiling_on_sc=False),
      scratch_types=scratch,
  )


def _tc_mm(x_ref, w_ref, p_ref):
  p_ref[...] = jnp.dot(x_ref[...], w_ref[...],
                       preferred_element_type=jnp.float32)


def _tc_scale(n, d3, dp_ref, p_ref, g_ref, dinv_ref):
  deg = dp_ref[:n, 0:1] + dp_ref[:n, d3:d3 + 1] + 1.0
  dinv = lax.rsqrt(jnp.maximum(deg, 1.0))
  g_ref[...] = dinv * p_ref[...]
  dinv_ref[...] = dinv


def _tc_mid(n, dh, ap_ref, g_ref, dinv_ref, b_ref, w_ref, gout_ref):
  dinv = dinv_ref[...]
  t = (dinv * (ap_ref[:n, :dh] + ap_ref[:n, dh:2 * dh] + g_ref[...])
       + b_ref[...])
  t = jnp.maximum(t, 0.0)
  gout_ref[...] = dinv * jnp.dot(t, w_ref[...],
                                 preferred_element_type=jnp.float32)


def _tc_tail(n, d3, do, ap_ref, g_ref, dinv_ref, b_ref, out_ref):
  t = (dinv_ref[...] * (ap_ref[:n, :d3] + ap_ref[:n, d3:2 * d3]
                        + g_ref[...]) + b_ref[...])
  out_ref[...] = t[:, :do]


def _splits(tchunks, ratio):
  s = tchunks // NS
  q0 = int(round(s * ratio / (1.0 + ratio) / NB)) * NB
  q0 = max(NB, min(q0, s - NB))
  q1 = ((s - q0) // NB) * NB
  rem = tchunks - NS * (q0 + q1)
  assert 0 <= rem <= NW and q0 >= NB and q1 >= NB, (tchunks, q0, q1, rem)
  return q0, q1, rem


def kernel(x, edge_index, W1, b1, W2, b2, W3, b3):
  n, d_in = x.shape
  e = edge_index.shape[1]
  dh = W1.shape[1]
  do = W3.shape[1]
  d3 = 8

  def chunked(cw):
    if e % cw == 0:
      ei = edge_index
    else:
      tail = cw - e % cw
      ei = jnp.concatenate(
          [edge_index,
           jnp.stack([jnp.zeros((tail,), edge_index.dtype),
                      jnp.full((tail,), n, edge_index.dtype)])], axis=1)
    tchunks = ei.shape[1] // cw
    return ei.reshape(2, tchunks, cw), tchunks

  ei3, tchunks = chunked(C)
  cw8 = 2 * C if e % (2 * C) == 0 else C
  ei3w, tchunksw = chunked(cw8)

  q0h, q1h, remh = _splits(tchunks, 1.55)
  q0s, q1s, rems = _splits(tchunksw, 1.0)

  nrows = -(-(n + 1) // (NS * 8)) * (NS * 8)

  z64 = jnp.zeros((nrows, dh), jnp.float32)
  z8 = jnp.zeros((nrows, d3), jnp.float32)
  ones8 = jnp.ones((cw8, d3), jnp.float32)
  W3p = jnp.concatenate([W3, jnp.zeros((dh, d3 - do), W3.dtype)], axis=1)
  b3p = jnp.concatenate([b3, jnp.zeros((d3 - do,), b3.dtype)])

  agg64 = _make_agg(dh, q0h, q1h, remh, nrows, n, "hybrid", C)
  agg8 = _make_agg(d3, q0s, q1s, rems, nrows, n, "spmem", cw8)
  deg8 = _make_agg(d3, q0s, q1s, rems, nrows, n, "const", cw8)

  degp = deg8(ones8, ei3w, z8)
  p1 = pl.pallas_call(
      _tc_mm, out_shape=jax.ShapeDtypeStruct((n, dh), jnp.float32),
  )(x, W1)

  g1, dinv = pl.pallas_call(
      functools.partial(_tc_scale, n, d3),
      out_shape=(jax.ShapeDtypeStruct((n, dh), jnp.float32),
                 jax.ShapeDtypeStruct((n, 1), jnp.float32)),
  )(degp, p1)

  a1 = agg64(g1, ei3, z64)
  g2 = pl.pallas_call(
      functools.partial(_tc_mid, n, dh),
      out_shape=jax.ShapeDtypeStruct((n, dh), jnp.float32),
  )(a1, g1, dinv, b1.reshape(1, dh), W2)

  a2 = agg64(g2, ei3, z64)
  g3 = pl.pallas_call(
      functools.partial(_tc_mid, n, dh),
      out_shape=jax.ShapeDtypeStruct((n, d3), jnp.float32),
  )(a2, g2, dinv, b2.reshape(1, dh), W3p)

  a3 = agg8(g3, ei3w, z8)
  dx = pl.pallas_call(
      functools.partial(_tc_tail, n, d3, do),
      out_shape=jax.ShapeDtypeStruct((n, do), jnp.float32),
  )(a3, g3, dinv, b3p.reshape(1, d3))

  return dx


# speedup vs baseline: 1.2474x; 1.0024x over previous
import functools

import jax
import jax.numpy as jnp
from jax import lax
from jax.experimental import pallas as pl
from jax.experimental.pallas import tpu as pltpu
from jax.experimental.pallas import tpu_sc as plsc

NC = 2
NS = 16
NW = NC * NS
C = 128
NB = 3
LA = 2
LANES = 128


def _agg_body(Q0, Q1, REM, RPT, NT, D, MODE, CW, g_hbm, ei3, z_hbm, out_hbm,
              acc, tbl, src_v, dst_v, *bufs):
  rows = bufs[:NB]
  gsems = bufs[NB:2 * NB]
  ssems = bufs[2 * NB:3 * NB]
  ones_v = bufs[3 * NB] if MODE == "const" else None
  cid = lax.axis_index("c")
  sid = lax.axis_index("s")
  wid = sid * NC + cid
  qmax = max(Q0, Q1)
  base = jnp.where(cid == 0, sid * Q0, NS * Q0 + sid * Q1)
  ebase = NS * (Q0 + Q1)

  @pl.when(cid == 0)
  def _():
    pltpu.sync_copy(ei3.at[0, pl.ds(base, Q0)], src_v.at[pl.ds(0, Q0)])
    pltpu.sync_copy(ei3.at[1, pl.ds(base, Q0)], dst_v.at[pl.ds(0, Q0)])

  @pl.when(cid == 1)
  def _():
    pltpu.sync_copy(ei3.at[0, pl.ds(base, Q1)], src_v.at[pl.ds(0, Q1)])
    pltpu.sync_copy(ei3.at[1, pl.ds(base, Q1)], dst_v.at[pl.ds(0, Q1)])

  @pl.when(wid < REM)
  def _():
    pltpu.sync_copy(ei3.at[0, pl.ds(ebase + wid, 1)],
                    src_v.at[pl.ds(qmax, 1)])
    pltpu.sync_copy(ei3.at[1, pl.ds(ebase + wid, 1)],
                    dst_v.at[pl.ds(qmax, 1)])

  pltpu.sync_copy(z_hbm.at[pl.ds(sid * RPT, RPT)],
                  acc.at[pl.ds(sid * RPT, RPT)])

  if MODE == "const":
    pltpu.sync_copy(g_hbm, ones_v)
  else:
    stage = (cid == 1) if MODE == "hybrid" else (cid >= 0)
    last = NT - (NS - 1) * RPT

    @pl.when(jnp.logical_and(stage, sid < NS - 1))
    def _():
      pltpu.sync_copy(g_hbm.at[pl.ds(sid * RPT, RPT)],
                      tbl.at[pl.ds(sid * RPT, RPT)])

    @pl.when(jnp.logical_and(stage, sid == NS - 1))
    def _():
      pltpu.sync_copy(g_hbm.at[pl.ds((NS - 1) * RPT, last)],
                      tbl.at[pl.ds((NS - 1) * RPT, last)])

  plsc.subcore_barrier()

  if MODE == "const":
    def cstep(c0, carry):
      for b in range(NB):
        c = c0 * NB + b

        @pl.when(c >= NB)
        def _():
          pltpu.make_async_copy(ones_v, acc.at[dst_v.at[c - NB]],
                                ssems[b]).wait()

        pltpu.async_copy(ones_v, acc.at[dst_v.at[c]], ssems[b], add=True)
      return carry

    lax.fori_loop(0, Q0 // NB, cstep, 0)
    for b in range(NB):
      pltpu.make_async_copy(ones_v, acc.at[dst_v.at[Q0 - NB + b]],
                            ssems[b]).wait()

    @pl.when(wid < REM)
    def _():
      pltpu.sync_copy(ones_v, acc.at[dst_v.at[qmax]], add=True)

  else:
    def pipeline(srcs, q):
      for j in range(LA):
        pltpu.async_copy(srcs[j].at[src_v.at[j]], rows[j], gsems[j])

      def step(c0, carry):
        for b in range(NB):
          c = c0 * NB + b
          f = c + LA
          bf = (b + LA) % NB

          @pl.when(jnp.logical_and(f < q, f >= NB))
          def _():
            pltpu.make_async_copy(rows[bf], acc.at[dst_v.at[f - NB]],
                                  ssems[bf]).wait()

          @pl.when(f < q)
          def _():
            pltpu.async_copy(srcs[bf].at[src_v.at[f]], rows[bf],
                             gsems[bf])

          pltpu.make_async_copy(srcs[b].at[src_v.at[c]], rows[b],
                                gsems[b]).wait()
          pltpu.async_copy(rows[b], acc.at[dst_v.at[c]], ssems[b],
                           add=True)
        return carry

      lax.fori_loop(0, q // NB, step, 0)
      for b in range(NB):
        pltpu.make_async_copy(rows[b], acc.at[dst_v.at[q - NB + b]],
                              ssems[b]).wait()

      @pl.when(wid < REM)
      def _():
        pltpu.async_copy(srcs[0].at[src_v.at[qmax]], rows[0],
                         gsems[0]).wait()
        pltpu.sync_copy(rows[0], acc.at[dst_v.at[qmax]], add=True)

    if MODE == "hybrid":
      @pl.when(cid == 0)
      def _():
        pipeline([g_hbm] * NB, Q0)

      @pl.when(cid == 1)
      def _():
        pipeline([tbl] * NB, Q1)
    else:
      pipeline([tbl] * NB, Q0)

  plsc.subcore_barrier()
  pltpu.sync_copy(acc.at[pl.ds(sid * RPT, RPT)],
                  out_hbm.at[pl.ds(sid * RPT, RPT), pl.ds(cid * D, D)])


@functools.cache
def _make_agg(d, q0, q1, rem, nrows, nt, mode, cw):
  rpt = nrows // NS
  qmax = max(q0, q1)
  assert q0 % NB == 0 and q1 % NB == 0, (q0, q1)
  mesh = plsc.VectorSubcoreMesh(core_axis_name="c", subcore_axis_name="s",
                                num_cores=NC, num_subcores=NS)
  scratch = [
      pltpu.VMEM_SHARED((nrows, d), jnp.float32),
      pltpu.VMEM_SHARED((nt, d), jnp.float32),
      pltpu.VMEM((qmax + 1, cw), jnp.int32),
      pltpu.VMEM((qmax + 1, cw), jnp.int32),
      *[pltpu.VMEM((cw, d), jnp.float32) for _ in range(NB)],
      *[pltpu.SemaphoreType.DMA for _ in range(2 * NB)],
  ]
  if mode == "const":
    scratch.append(pltpu.VMEM((cw, d), jnp.float32))
  return pl.kernel(
      functools.partial(_agg_body, q0, q1, rem, rpt, nt, d, mode, cw),
      out_type=jax.ShapeDtypeStruct((nrows, LANES), jnp.float32),
      mesh=mesh,
      compiler_params=pltpu.CompilerParams(use_tc_tiling_on_sc=False),
      scratch_types=scratch,
  )


def _tc_mm(x_ref, w_ref, p_ref):
  p_ref[...] = jnp.dot(x_ref[...], w_ref[...],
                       preferred_element_type=jnp.float32)


def _tc_scale(n, d3, dp_ref, p_ref, g_ref, dinv_ref):
  deg = dp_ref[:n, 0:1] + dp_ref[:n, d3:d3 + 1] + 1.0
  dinv = lax.rsqrt(jnp.maximum(deg, 1.0))
  g_ref[...] = dinv * p_ref[...]
  dinv_ref[...] = dinv


def _tc_mid(n, dh, ap_ref, g_ref, dinv_ref, b_ref, w_ref, gout_ref):
  dinv = dinv_ref[...]
  t = (dinv * (ap_ref[:n, :dh] + ap_ref[:n, dh:2 * dh] + g_ref[...])
       + b_ref[...])
  t = jnp.maximum(t, 0.0)
  gout_ref[...] = dinv * jnp.dot(t, w_ref[...],
                                 preferred_element_type=jnp.float32)


def _tc_tail(n, d3, do, ap_ref, g_ref, dinv_ref, b_ref, out_ref):
  t = (dinv_ref[...] * (ap_ref[:n, :d3] + ap_ref[:n, d3:2 * d3]
                        + g_ref[...]) + b_ref[...])
  out_ref[...] = t[:, :do]


def _splits(tchunks, ratio):
  s = tchunks // NS
  q0 = int(round(s * ratio / (1.0 + ratio) / NB)) * NB
  q0 = max(NB, min(q0, s - NB))
  q1 = ((s - q0) // NB) * NB
  rem = tchunks - NS * (q0 + q1)
  assert 0 <= rem <= NW and q0 >= NB and q1 >= NB, (tchunks, q0, q1, rem)
  return q0, q1, rem


def kernel(x, edge_index, W1, b1, W2, b2, W3, b3):
  n, d_in = x.shape
  e = edge_index.shape[1]
  dh = W1.shape[1]
  do = W3.shape[1]
  d3 = 8

  def chunked(cw):
    if e % cw == 0:
      ei = edge_index
    else:
      tail = cw - e % cw
      ei = jnp.concatenate(
          [edge_index,
           jnp.stack([jnp.zeros((tail,), edge_index.dtype),
                      jnp.full((tail,), n, edge_index.dtype)])], axis=1)
    tchunks = ei.shape[1] // cw
    return ei.reshape(2, tchunks, cw), tchunks

  ei3, tchunks = chunked(C)
  cw8 = 2 * C if e % (2 * C) == 0 else C
  ei3w, tchunksw = chunked(cw8)

  q0h, q1h, remh = _splits(tchunks, 1.55)
  q0s, q1s, rems = _splits(tchunksw, 1.0)

  nrows = -(-(n + 1) // (NS * 8)) * (NS * 8)

  z64 = jnp.zeros((nrows, dh), jnp.float32)
  z8 = jnp.zeros((nrows, d3), jnp.float32)
  ones8 = jnp.ones((cw8, d3), jnp.float32)
  W3p = jnp.concatenate([W3, jnp.zeros((dh, d3 - do), W3.dtype)], axis=1)
  b3p = jnp.concatenate([b3, jnp.zeros((d3 - do,), b3.dtype)])

  agg64 = _make_agg(dh, q0h, q1h, remh, nrows, n, "hybrid", C)
  agg8 = _make_agg(d3, q0s, q1s, rems, nrows, n, "spmem", cw8)
  deg8 = _make_agg(d3, q0s, q1s, rems, nrows, n, "const", cw8)

  blk = n
  for cand in (2000, 2500, 1250, 1000):
    if n % cand == 0 and cand % 8 == 0:
      blk = cand
      break
  nb = n // blk

  def row(bs):
    return pl.BlockSpec((blk, bs), lambda i: (i, 0))

  def full(a, b):
    return pl.BlockSpec((a, b), lambda i: (0, 0))

  degp = deg8(ones8, ei3w, z8)
  p1 = pl.pallas_call(
      _tc_mm, out_shape=jax.ShapeDtypeStruct((n, dh), jnp.float32),
      grid=(nb,), in_specs=[row(d_in), full(d_in, dh)],
      out_specs=row(dh),
  )(x, W1)

  g1, dinv = pl.pallas_call(
      functools.partial(_tc_scale, blk, d3),
      out_shape=(jax.ShapeDtypeStruct((n, dh), jnp.float32),
                 jax.ShapeDtypeStruct((n, 1), jnp.float32)),
      grid=(nb,), in_specs=[row(LANES), row(dh)],
      out_specs=(row(dh), row(1)),
  )(degp, p1)

  a1 = agg64(g1, ei3, z64)
  g2 = pl.pallas_call(
      functools.partial(_tc_mid, blk, dh),
      out_shape=jax.ShapeDtypeStruct((n, dh), jnp.float32),
      grid=(nb,),
      in_specs=[row(LANES), row(dh), row(1), full(1, dh), full(dh, dh)],
      out_specs=row(dh),
  )(a1, g1, dinv, b1.reshape(1, dh), W2)

  a2 = agg64(g2, ei3, z64)
  g3 = pl.pallas_call(
      functools.partial(_tc_mid, blk, dh),
      out_shape=jax.ShapeDtypeStruct((n, d3), jnp.float32),
      grid=(nb,),
      in_specs=[row(LANES), row(dh), row(1), full(1, dh), full(dh, d3)],
      out_specs=row(d3),
  )(a2, g2, dinv, b2.reshape(1, dh), W3p)

  a3 = agg8(g3, ei3w, z8)
  dx = pl.pallas_call(
      functools.partial(_tc_tail, blk, d3, do),
      out_shape=jax.ShapeDtypeStruct((n, do), jnp.float32),
      grid=(nb,),
      in_specs=[row(LANES), row(d3), row(1), full(1, d3)],
      out_specs=row(do),
  )(a3, g3, dinv, b3p.reshape(1, d3))

  return dx
